# baseline (speedup 1.0000x reference)
import functools

import jax
import jax.numpy as jnp
from jax import lax
from jax.experimental import pallas as pl
from jax.experimental.pallas import tpu as pltpu
from jax.experimental.pallas import tpu_sc as plsc

_ROWS = 1000000
_D = 32
_B = 16384
_F = 26
_TOTAL = _B * _F
_NC = 2
_NS = 16
_NW = _NC * _NS
_PW = _TOTAL // _NW

_mesh = plsc.VectorSubcoreMesh(core_axis_name="c", subcore_axis_name="s")


@functools.partial(
    pl.kernel,
    out_type=jax.ShapeDtypeStruct((_TOTAL,), jnp.int32),
    mesh=_mesh,
    scratch_types=[pltpu.VMEM((_PW,), jnp.int32)],
    compiler_params=pltpu.CompilerParams(use_tc_tiling_on_sc=True),
)
def _xprep(xt_hbm, out_hbm, buf):
    wid = lax.axis_index("s") * _NC + lax.axis_index("c")
    for w in range(_NW):
        @pl.when(wid == w)
        def _():
            base = w * _PW
            f0, b0 = divmod(base, _B)
            n0 = min(_B - b0, _PW)
            pltpu.sync_copy(xt_hbm.at[f0, pl.ds(b0, n0)], buf.at[pl.ds(0, n0)])
            if n0 < _PW:
                pltpu.sync_copy(
                    xt_hbm.at[f0 + 1, pl.ds(0, _PW - n0)],
                    buf.at[pl.ds(n0, _PW - n0)],
                )
    pltpu.sync_copy(buf, out_hbm.at[pl.ds(wid * _PW, _PW)])


_C2PW = 62
_TAIL = _ROWS - 7812 * 128


@functools.partial(
    pl.kernel,
    out_type=jax.ShapeDtypeStruct((_ROWS * _D,), jnp.float32),
    mesh=_mesh,
    scratch_types=[
        pltpu.VMEM((32, 512), jnp.float32),
        pltpu.VMEM((32, 512), jnp.float32),
        pltpu.VMEM((512 * _D,), jnp.float32),
        pltpu.VMEM((512 * _D,), jnp.float32),
        pltpu.VMEM((_TAIL * _D,), jnp.float32),
        pltpu.SemaphoreType.DMA,
        pltpu.SemaphoreType.DMA,
        pltpu.SemaphoreType.DMA,
        pltpu.SemaphoreType.DMA,
    ],
    compiler_params=pltpu.CompilerParams(
        use_tc_tiling_on_sc=True, needs_layout_passes=False
    ),
)
def _ttrans(tt_hbm, tail_hbm, tlin_hbm, inb0, inb1, trb0, trb1, tbuf,
            is0, is1, os0, os1):
    wid = lax.axis_index("s") * _NC + lax.axis_index("c")
    lanes = jnp.arange(16, dtype=jnp.int32)
    ar = jnp.arange(16)
    rowv = [[(ar + s) % 16 + d0 for s in range(16)] for d0 in (0, 16)]
    pv = [ar * _D + (ar + s) % 16 for s in range(16)]
    bufs = [(inb0, trb0, is0, os0), (inb1, trb1, is1, os1)]

    @pl.when(wid == _NW - 1)
    def _():
        pltpu.sync_copy(tail_hbm, tbuf)
        pltpu.sync_copy(tbuf, tlin_hbm.at[pl.ds(7812 * 128 * _D, _TAIL * _D)])

    def cstart(g):
        return jnp.minimum(4 * g, 7808) * 128

    def fire_in(g, inb, sem):
        c = cstart(g)
        for tr in range(4):
            pltpu.async_copy(
                tt_hbm.at[pl.ds(8 * tr, 8), pl.ds(c, 512)],
                inb.at[pl.ds(tr * 8, 8), :], sem,
            )

    def drain_in(inb, sem):
        for tr in range(4):
            pltpu.make_async_copy(
                tt_hbm.at[pl.ds(0, 8), pl.ds(0, 512)],
                inb.at[pl.ds(tr * 8, 8), :], sem,
            ).wait()

    def compute(inb, trb):
        def jloop(jb, carry):
            jv = lanes + jb * 16
            jsplat = jnp.zeros((16,), jnp.int32) + jb * (16 * _D)
            for tcs in range(4):
                cvt = jv + tcs * 128
                for di, d0 in enumerate((0, 16)):
                    for s in range(16):
                        g2 = plsc.load_gather(inb, [rowv[di][s], cvt])
                        pos = (pv[s] + (tcs * 4096 + d0)).astype(jnp.int32)
                        plsc.store_scatter(trb, [pos + jsplat], g2)
            return carry
        lax.fori_loop(0, 8, jloop, 0)

    def drain_out(trb, sem):
        pltpu.make_async_copy(trb, tlin_hbm.at[pl.ds(0, 512 * _D)], sem).wait()

    for p in range(2):
        fire_in(wid * _C2PW + p, bufs[p][0], bufs[p][2])

    def body(m, carry):
        for p in range(2):
            inb, trb, isem, osem = bufs[p]
            g = wid * _C2PW + 2 * m + p
            drain_in(inb, isem)
            @pl.when(m >= 1)
            def _drain(trb=trb, osem=osem):
                drain_out(trb, osem)
            compute(inb, trb)
            pltpu.async_copy(trb, tlin_hbm.at[pl.ds(cstart(g) * _D, 512 * _D)], osem)
            @pl.when(2 * m + p + 2 < _C2PW)
            def _fire(g=g, inb=inb, isem=isem):
                fire_in(g + 2, inb, isem)
        return carry

    lax.fori_loop(0, _C2PW // 2, body, 0)
    for p in range(2):
        drain_out(bufs[p][1], bufs[p][3])


_QB = 512
_NQ = _B // _QB
_UNITS_PW = _F * _NQ // _NW


@functools.partial(
    pl.kernel,
    out_type=jax.ShapeDtypeStruct((_TOTAL * _D,), jnp.float32),
    mesh=_mesh,
    scratch_types=[
        pltpu.VMEM((_QB,), jnp.int32),
        pltpu.VMEM((_QB,), jnp.int32),
        pltpu.VMEM((_QB, _D), jnp.float32),
        pltpu.VMEM((_QB, _D), jnp.float32),
        pltpu.VMEM((_QB * _D,), jnp.float32),
        pltpu.VMEM((_QB * _D,), jnp.float32),
        pltpu.SemaphoreType.DMA,
        pltpu.SemaphoreType.DMA,
        pltpu.SemaphoreType.DMA,
        pltpu.SemaphoreType.DMA,
    ],
    compiler_params=pltpu.CompilerParams(
        use_tc_tiling_on_sc=False, needs_layout_passes=False
    ),
)
def _gather5(idx_hbm, table_hbm, out_hbm, iv0, iv1, gb0, gb1, ob0, ob1,
             gs0, gs1, os0, os1):
    wid = lax.axis_index("s") * _NC + lax.axis_index("c")
    base = wid * _UNITS_PW
    lanes = jnp.arange(16, dtype=jnp.int32)
    colv = [(lanes + s) % 16 for s in range(16)]
    dd = [(jnp.arange(16) + s) % 16 for s in range(16)]
    q5 = [(d // 8) * 4096 + (d % 8) * 128 + jnp.arange(16) for d in dd]
    bufs = [(iv0, gb0, ob0, gs0, os0), (iv1, gb1, ob1, gs1, os1)]

    def stage(u, iv, gsem):
        f = u // _NQ
        q = u % _NQ
        pltpu.sync_copy(idx_hbm.at[pl.ds(f * _B + q * _QB, _QB)], iv)
        pltpu.async_copy(table_hbm.at[iv], bufs[0][1] if iv is iv0 else bufs[1][1], gsem)

    def compute(gbuf, obuf):
        def jloop(jb, carry):
            rowv = lanes + jb * 16
            jsplat = jnp.zeros((16,), jnp.int32) + ((jb // 8) * 1024 + (jb % 8) * 16)
            for d0 in (0, 16):
                for s in range(16):
                    g = plsc.load_gather(gbuf, [rowv, colv[s] + d0])
                    pos = (q5[s] + (d0 // 8) * 4096).astype(jnp.int32)
                    plsc.store_scatter(obuf, [pos + jsplat], g)
            return carry
        lax.fori_loop(0, _QB // 16, jloop, 0)

    def fire_out(u, obuf, osem):
        f = u // _NQ
        q = u % _NQ
        for tr in range(4):
            pltpu.async_copy(
                obuf.at[pl.ds(tr * 4096, 4096)],
                out_hbm.at[pl.ds(((f * 4 + tr) * 128 + q * 4) * 1024, 4096)],
                osem,
            )

    for p in range(2):
        stage(base + p, bufs[p][0], bufs[p][3])

    def body(m, carry):
        for p in range(2):
            iv, gbuf, obuf, gsem, osem = bufs[p]
            u = base + 2 * m + p
            pltpu.make_async_copy(
                table_hbm.at[pl.ds(0, _QB)], gbuf, gsem
            ).wait()
            @pl.when(m >= 1)
            def _dr(obuf=obuf, osem=osem):
                for _ in range(4):
                    pltpu.make_async_copy(
                        obuf.at[pl.ds(0, 4096)],
                        out_hbm.at[pl.ds(0, 4096)], osem,
                    ).wait()
            compute(gbuf, obuf)
            fire_out(u, obuf, osem)
            @pl.when(2 * m + p + 2 < _UNITS_PW)
            def _pf(u=u, iv=iv, gsem=gsem):
                stage(u + 2, iv, gsem)
        return carry

    lax.fori_loop(0, _UNITS_PW // 2, body, 0)
    for p in range(2):
        for _ in range(4):
            pltpu.make_async_copy(
                bufs[p][2].at[pl.ds(0, 4096)],
                out_hbm.at[pl.ds(0, 4096)], bufs[p][4],
            ).wait()


def kernel(x, table):
    xt = x.T.astype(jnp.int32)
    tt = table.T
    tail = lax.slice(table, (7812 * 128, 0), (_ROWS, _D)).reshape(-1)
    idx = _xprep(xt)
    tlin = _ttrans(tt, tail)
    o5 = _gather5(idx, tlin.reshape(_ROWS, _D))
    return (
        o5.reshape(_F, 4, 128, 8, 128)
        .transpose(2, 4, 0, 1, 3)
        .reshape(_B, _F, _D)
    )


# speedup vs baseline: 1.0149x; 1.0149x over previous
import functools

import jax
import jax.numpy as jnp
from jax import lax
from jax.experimental import pallas as pl
from jax.experimental.pallas import tpu as pltpu
from jax.experimental.pallas import tpu_sc as plsc

_ROWS = 1000000
_D = 32
_B = 16384
_F = 26
_TOTAL = _B * _F
_NC = 2
_NS = 16
_NW = _NC * _NS
_PW = _TOTAL // _NW

_mesh = plsc.VectorSubcoreMesh(core_axis_name="c", subcore_axis_name="s")


@functools.partial(
    pl.kernel,
    out_type=jax.ShapeDtypeStruct((_TOTAL,), jnp.int32),
    mesh=_mesh,
    scratch_types=[pltpu.VMEM((_PW,), jnp.int32)],
    compiler_params=pltpu.CompilerParams(use_tc_tiling_on_sc=True),
)
def _xprep(xt_hbm, out_hbm, buf):
    wid = lax.axis_index("s") * _NC + lax.axis_index("c")
    for w in range(_NW):
        @pl.when(wid == w)
        def _():
            base = w * _PW
            f0, b0 = divmod(base, _B)
            n0 = min(_B - b0, _PW)
            pltpu.sync_copy(xt_hbm.at[f0, pl.ds(b0, n0)], buf.at[pl.ds(0, n0)])
            if n0 < _PW:
                pltpu.sync_copy(
                    xt_hbm.at[f0 + 1, pl.ds(0, _PW - n0)],
                    buf.at[pl.ds(n0, _PW - n0)],
                )
    pltpu.sync_copy(buf, out_hbm.at[pl.ds(wid * _PW, _PW)])


_C2PW = 62
_TAIL = _ROWS - 7812 * 128


@functools.partial(
    pl.kernel,
    out_type=jax.ShapeDtypeStruct((_ROWS * _D,), jnp.float32),
    mesh=_mesh,
    scratch_types=[
        pltpu.VMEM((128, 128), jnp.float32),
        pltpu.VMEM((128, 128), jnp.float32),
        pltpu.VMEM((512 * _D,), jnp.float32),
        pltpu.VMEM((512 * _D,), jnp.float32),
        pltpu.VMEM((_TAIL * _D,), jnp.float32),
        pltpu.SemaphoreType.DMA,
        pltpu.SemaphoreType.DMA,
        pltpu.SemaphoreType.DMA,
        pltpu.SemaphoreType.DMA,
    ],
    compiler_params=pltpu.CompilerParams(
        use_tc_tiling_on_sc=True, needs_layout_passes=False
    ),
)
def _ttrans(tt_hbm, tail_hbm, tlin_hbm, inb0, inb1, trb0, trb1, tbuf,
            is0, is1, os0, os1):
    wid = lax.axis_index("s") * _NC + lax.axis_index("c")
    lanes = jnp.arange(16, dtype=jnp.int32)
    ar = jnp.arange(16)
    rowv = [[[(ar + s) % 16 + (tcs * 32 + d0) for s in range(16)]
             for d0 in (0, 16)] for tcs in range(4)]
    pv = [ar * _D + (ar + s) % 16 for s in range(16)]
    bufs = [(inb0, trb0, is0, os0), (inb1, trb1, is1, os1)]

    @pl.when(wid == _NW - 1)
    def _():
        pltpu.sync_copy(tail_hbm, tbuf)
        pltpu.sync_copy(tbuf, tlin_hbm.at[pl.ds(7812 * 128 * _D, _TAIL * _D)])

    def cstart(g):
        return jnp.minimum(4 * g, 7808) * 128

    def fire_in(g, inb, sem):
        c = cstart(g)
        for tcs in range(4):
            for tr in range(4):
                pltpu.async_copy(
                    tt_hbm.at[pl.ds(8 * tr, 8), pl.ds(c + tcs * 128, 128)],
                    inb.at[pl.ds((tcs * 4 + tr) * 8, 8), :], sem,
                )

    def drain_in(inb, sem):
        for k in range(16):
            pltpu.make_async_copy(
                tt_hbm.at[pl.ds(0, 8), pl.ds(0, 128)],
                inb.at[pl.ds(8 * k, 8), :], sem,
            ).wait()

    def compute(inb, trb):
        def jloop(jb, carry):
            jv = lanes + jb * 16
            jsplat = jnp.zeros((16,), jnp.int32) + jb * (16 * _D)
            for tcs in range(4):
                for di, d0 in enumerate((0, 16)):
                    for s in range(16):
                        g2 = plsc.load_gather(inb, [rowv[tcs][di][s], jv])
                        pos = (pv[s] + (tcs * 4096 + d0)).astype(jnp.int32)
                        plsc.store_scatter(trb, [pos + jsplat], g2)
            return carry
        lax.fori_loop(0, 8, jloop, 0)

    def drain_out(trb, sem):
        pltpu.make_async_copy(trb, tlin_hbm.at[pl.ds(0, 512 * _D)], sem).wait()

    for p in range(2):
        fire_in(wid * _C2PW + p, bufs[p][0], bufs[p][2])

    def body(m, carry):
        for p in range(2):
            inb, trb, isem, osem = bufs[p]
            g = wid * _C2PW + 2 * m + p
            drain_in(inb, isem)
            @pl.when(m >= 1)
            def _drain(trb=trb, osem=osem):
                drain_out(trb, osem)
            compute(inb, trb)
            pltpu.async_copy(trb, tlin_hbm.at[pl.ds(cstart(g) * _D, 512 * _D)], osem)
            @pl.when(2 * m + p + 2 < _C2PW)
            def _fire(g=g, inb=inb, isem=isem):
                fire_in(g + 2, inb, isem)
        return carry

    lax.fori_loop(0, _C2PW // 2, body, 0)
    for p in range(2):
        drain_out(bufs[p][1], bufs[p][3])


_QB = 512
_NQ = _B // _QB
_UNITS_PW = _F * _NQ // _NW


@functools.partial(
    pl.kernel,
    out_type=jax.ShapeDtypeStruct((_TOTAL * _D,), jnp.float32),
    mesh=_mesh,
    scratch_types=[
        pltpu.VMEM((_QB,), jnp.int32),
        pltpu.VMEM((_QB,), jnp.int32),
        pltpu.VMEM((_QB, _D), jnp.float32),
        pltpu.VMEM((_QB, _D), jnp.float32),
        pltpu.VMEM((_QB * _D,), jnp.float32),
        pltpu.VMEM((_QB * _D,), jnp.float32),
        pltpu.SemaphoreType.DMA,
        pltpu.SemaphoreType.DMA,
        pltpu.SemaphoreType.DMA,
        pltpu.SemaphoreType.DMA,
    ],
    compiler_params=pltpu.CompilerParams(
        use_tc_tiling_on_sc=False, needs_layout_passes=False
    ),
)
def _gather5(idx_hbm, table_hbm, out_hbm, iv0, iv1, gb0, gb1, ob0, ob1,
             gs0, gs1, os0, os1):
    wid = lax.axis_index("s") * _NC + lax.axis_index("c")
    base = wid * _UNITS_PW
    lanes = jnp.arange(16, dtype=jnp.int32)
    colv = [(lanes + s) % 16 for s in range(16)]
    dd = [(jnp.arange(16) + s) % 16 for s in range(16)]
    q5 = [(d // 8) * 4096 + (d % 8) * 128 + jnp.arange(16) for d in dd]
    bufs = [(iv0, gb0, ob0, gs0, os0), (iv1, gb1, ob1, gs1, os1)]

    def stage(u, iv, gsem):
        f = u // _NQ
        q = u % _NQ
        pltpu.sync_copy(idx_hbm.at[pl.ds(f * _B + q * _QB, _QB)], iv)
        pltpu.async_copy(table_hbm.at[iv], bufs[0][1] if iv is iv0 else bufs[1][1], gsem)

    def compute(gbuf, obuf):
        def jloop(jb, carry):
            rowv = lanes + jb * 16
            jsplat = jnp.zeros((16,), jnp.int32) + ((jb // 8) * 1024 + (jb % 8) * 16)
            for d0 in (0, 16):
                for s in range(16):
                    g = plsc.load_gather(gbuf, [rowv, colv[s] + d0])
                    pos = (q5[s] + (d0 // 8) * 4096).astype(jnp.int32)
                    plsc.store_scatter(obuf, [pos + jsplat], g)
            return carry
        lax.fori_loop(0, _QB // 16, jloop, 0)

    def fire_out(u, obuf, osem):
        f = u // _NQ
        q = u % _NQ
        for tr in range(4):
            pltpu.async_copy(
                obuf.at[pl.ds(tr * 4096, 4096)],
                out_hbm.at[pl.ds(((f * 4 + tr) * 128 + q * 4) * 1024, 4096)],
                osem,
            )

    for p in range(2):
        stage(base + p, bufs[p][0], bufs[p][3])

    def body(m, carry):
        for p in range(2):
            iv, gbuf, obuf, gsem, osem = bufs[p]
            u = base + 2 * m + p
            pltpu.make_async_copy(
                table_hbm.at[pl.ds(0, _QB)], gbuf, gsem
            ).wait()
            @pl.when(m >= 1)
            def _dr(obuf=obuf, osem=osem):
                for _ in range(4):
                    pltpu.make_async_copy(
                        obuf.at[pl.ds(0, 4096)],
                        out_hbm.at[pl.ds(0, 4096)], osem,
                    ).wait()
            compute(gbuf, obuf)
            fire_out(u, obuf, osem)
            @pl.when(2 * m + p + 2 < _UNITS_PW)
            def _pf(u=u, iv=iv, gsem=gsem):
                stage(u + 2, iv, gsem)
        return carry

    lax.fori_loop(0, _UNITS_PW // 2, body, 0)
    for p in range(2):
        for _ in range(4):
            pltpu.make_async_copy(
                bufs[p][2].at[pl.ds(0, 4096)],
                out_hbm.at[pl.ds(0, 4096)], bufs[p][4],
            ).wait()


def kernel(x, table):
    xt = x.T.astype(jnp.int32)
    tt = table.T
    tail = lax.slice(table, (7812 * 128, 0), (_ROWS, _D)).reshape(-1)
    idx = _xprep(xt)
    tlin = _ttrans(tt, tail)
    o5 = _gather5(idx, tlin.reshape(_ROWS, _D))
    return (
        o5.reshape(_F, 4, 128, 8, 128)
        .transpose(2, 4, 0, 1, 3)
        .reshape(_B, _F, _D)
    )
